# Initial kernel scaffold; baseline (speedup 1.0000x reference)
import jax
import jax.numpy as jnp
from jax.experimental import pallas as pl


def kernel(x, edge_index, W1, b1, W2, b2):
    raise NotImplementedError("write your pallas kernel here")


# speedup vs baseline: 12.1624x; 12.1624x over previous
import functools

import jax
import jax.numpy as jnp
from jax import lax
from jax.experimental import pallas as pl
from jax.experimental.pallas import tpu as pltpu
from jax.experimental.pallas import tpu_sc as plsc

N_NODES = 10000
NPAD = 10240
D_IN = 128
D_HID = 128
D_OUT = 64
N_EDGES = 320000
NW = 32
CH = 128
EPW = ((N_EDGES + NW * CH - 1) // (NW * CH)) * CH
EPAD = NW * EPW
ROWS_PER_TILE = NPAD // 16

_mesh = plsc.VectorSubcoreMesh(core_axis_name="c", subcore_axis_name="s")


@functools.partial(
    pl.kernel,
    out_type=jax.ShapeDtypeStruct((2 * NPAD,), jnp.float32),
    mesh=_mesh,
    scratch_types=[
        pltpu.VMEM((CH,), jnp.int32),
        pltpu.VMEM((CH,), jnp.float32),
        pltpu.VMEM_SHARED((NPAD,), jnp.float32),
    ],
)
def _sc_count(dst_hbm, zeros1_hbm, out_hbm, dst_v, ones_v, acc):
    c = lax.axis_index("c")
    s = lax.axis_index("s")
    wid = s * 2 + c
    for j in range(CH // 16):
        ones_v[pl.ds(j * 16, 16)] = jnp.ones((16,), jnp.float32)
    rbase = s * ROWS_PER_TILE
    pltpu.sync_copy(zeros1_hbm.at[pl.ds(rbase, ROWS_PER_TILE)],
                    acc.at[pl.ds(rbase, ROWS_PER_TILE)])
    plsc.subcore_barrier()

    def body(i, carry):
        off = wid * EPW + i * CH
        pltpu.sync_copy(dst_hbm.at[pl.ds(off, CH)], dst_v)
        pltpu.sync_copy(ones_v, acc.at[dst_v], add=True)
        return carry

    lax.fori_loop(0, EPW // CH, body, 0)
    plsc.subcore_barrier()
    pltpu.sync_copy(acc.at[pl.ds(rbase, ROWS_PER_TILE)],
                    out_hbm.at[pl.ds(c * NPAD + rbase, ROWS_PER_TILE)])


def _make_sc_scatter(D):
    params = (pltpu.CompilerParams(use_tc_tiling_on_sc=False)
              if D % 128 != 0 else None)

    @functools.partial(
        pl.kernel,
        out_type=jax.ShapeDtypeStruct((2 * NPAD, D), jnp.float32),
        mesh=_mesh,
        compiler_params=params,
        scratch_types=[
            pltpu.VMEM((CH,), jnp.int32),
            pltpu.VMEM((CH,), jnp.int32),
            pltpu.VMEM((CH, D), jnp.float32),
            pltpu.VMEM_SHARED((NPAD, D), jnp.float32),
            pltpu.SemaphoreType.DMA,
        ],
    )
    def k(hs_hbm, src_hbm, dst_hbm, zeros_hbm, out_hbm,
          src_v, dst_v, rows_v, acc, sem):
        c = lax.axis_index("c")
        s = lax.axis_index("s")
        wid = s * 2 + c
        rbase = s * ROWS_PER_TILE
        pltpu.sync_copy(zeros_hbm.at[pl.ds(rbase, ROWS_PER_TILE)],
                        acc.at[pl.ds(rbase, ROWS_PER_TILE)])
        plsc.subcore_barrier()

        def body(i, carry):
            off = wid * EPW + i * CH
            pltpu.sync_copy(src_hbm.at[pl.ds(off, CH)], src_v)
            pltpu.sync_copy(dst_hbm.at[pl.ds(off, CH)], dst_v)
            pltpu.async_copy(hs_hbm.at[src_v], rows_v, sem).wait()
            pltpu.sync_copy(rows_v, acc.at[dst_v], add=True)
            return carry

        lax.fori_loop(0, EPW // CH, body, 0)
        plsc.subcore_barrier()
        pltpu.sync_copy(acc.at[pl.ds(rbase, ROWS_PER_TILE)],
                        out_hbm.at[pl.ds(c * NPAD + rbase, ROWS_PER_TILE)])

    return k


_sc_scatter_hid = _make_sc_scatter(D_HID)
_sc_scatter_out = _make_sc_scatter(D_OUT)


BN = 512


def _tc1_body(cnt_ref, x_ref, w_ref, dis_ref, hs_ref):
    cnt = cnt_ref[:, 0:1] + cnt_ref[:, 1:2]
    dis = lax.rsqrt(cnt + 1.0)
    dis_ref[...] = dis
    h = jnp.dot(x_ref[...], w_ref[...], preferred_element_type=jnp.float32)
    hs_ref[...] = h * dis


def _tc1(cnt2, x_p, W1):
    grid = NPAD // BN
    return pl.pallas_call(
        _tc1_body,
        grid=(grid,),
        in_specs=[
            pl.BlockSpec((BN, 2), lambda i: (i, 0)),
            pl.BlockSpec((BN, D_IN), lambda i: (i, 0)),
            pl.BlockSpec((D_IN, D_HID), lambda i: (0, 0)),
        ],
        out_specs=[
            pl.BlockSpec((BN, 1), lambda i: (i, 0)),
            pl.BlockSpec((BN, D_HID), lambda i: (i, 0)),
        ],
        out_shape=[
            jax.ShapeDtypeStruct((NPAD, 1), jnp.float32),
            jax.ShapeDtypeStruct((NPAD, D_HID), jnp.float32),
        ],
    )(cnt2, x_p, W1)


def _tc2_body(p_ref, hs_ref, dis_ref, b_ref, w_ref, hs2_ref):
    agg = p_ref[0] + p_ref[1] + hs_ref[...]
    dis = dis_ref[...]
    h1 = jnp.maximum(agg * dis + b_ref[...], 0.0)
    hs2_ref[...] = jnp.dot(h1, w_ref[...],
                           preferred_element_type=jnp.float32) * dis


def _tc2(p, hs1, dis, b1r, W2):
    grid = NPAD // BN
    return pl.pallas_call(
        _tc2_body,
        grid=(grid,),
        in_specs=[
            pl.BlockSpec((2, BN, D_HID), lambda i: (0, i, 0)),
            pl.BlockSpec((BN, D_HID), lambda i: (i, 0)),
            pl.BlockSpec((BN, 1), lambda i: (i, 0)),
            pl.BlockSpec((1, D_HID), lambda i: (0, 0)),
            pl.BlockSpec((D_HID, D_OUT), lambda i: (0, 0)),
        ],
        out_specs=pl.BlockSpec((BN, D_OUT), lambda i: (i, 0)),
        out_shape=jax.ShapeDtypeStruct((NPAD, D_OUT), jnp.float32),
    )(p, hs1, dis, b1r, W2)


def _tc3_body(q_ref, hs2_ref, dis_ref, b_ref, out_ref):
    agg = q_ref[0] + q_ref[1] + hs2_ref[...]
    out_ref[...] = agg * dis_ref[...] + b_ref[...]


def _tc3(q, hs2, dis, b2r):
    grid = NPAD // BN
    return pl.pallas_call(
        _tc3_body,
        grid=(grid,),
        in_specs=[
            pl.BlockSpec((2, BN, D_OUT), lambda i: (0, i, 0)),
            pl.BlockSpec((BN, D_OUT), lambda i: (i, 0)),
            pl.BlockSpec((BN, 1), lambda i: (i, 0)),
            pl.BlockSpec((1, D_OUT), lambda i: (0, 0)),
        ],
        out_specs=pl.BlockSpec((BN, D_OUT), lambda i: (i, 0)),
        out_shape=jax.ShapeDtypeStruct((NPAD, D_OUT), jnp.float32),
    )(q, hs2, dis, b2r)


def kernel(x, edge_index, W1, b1, W2, b2):
    src = edge_index[0]
    dst = edge_index[1]
    pad_e = EPAD - N_EDGES
    src_p = jnp.concatenate([src, jnp.zeros((pad_e,), jnp.int32)])
    dst_p = jnp.concatenate([dst, jnp.full((pad_e,), N_NODES, jnp.int32)])
    x_p = jnp.concatenate(
        [x, jnp.zeros((NPAD - N_NODES, D_IN), jnp.float32)], axis=0)
    zeros1 = jnp.zeros((NPAD,), jnp.float32)
    zeros_hid = jnp.zeros((NPAD, D_HID), jnp.float32)
    zeros_out = jnp.zeros((NPAD, D_OUT), jnp.float32)

    cnt = _sc_count(dst_p, zeros1)
    cnt2 = cnt.reshape(2, NPAD).T
    dis, hs1 = _tc1(cnt2, x_p, W1)
    p = _sc_scatter_hid(hs1, src_p, dst_p, zeros_hid).reshape(2, NPAD, D_HID)
    hs2 = _tc2(p, hs1, dis, b1.reshape(1, D_HID), W2)
    q = _sc_scatter_out(hs2, src_p, dst_p, zeros_out).reshape(2, NPAD, D_OUT)
    out = _tc3(q, hs2, dis, b2.reshape(1, D_OUT))
    return out[:N_NODES]
